# baseline (speedup 1.0000x reference)
import jax
import jax.numpy as jnp
from jax.experimental import pallas as pl
from jax.experimental.pallas import tpu as pltpu

BASE = 10000.0
NB = 2048
ROW = 1536
NT, NY, NX = 16, 32, 32


NBI = 256


def _rope_kernel(pos_ref, tab_ref, tokens_ref, out_ref):
    tab = tab_ref[...]
    for j in range(NB // NBI):
        sl = pl.ds(j * NBI, NBI)
        p = pos_ref[:, sl, :].astype(jnp.int32)
        l80 = jax.lax.broadcasted_iota(jnp.int32, (1, NBI, NT + NY + NX), 2)
        hit = (l80 == p[:, :, 0:1]) | (l80 == p[:, :, 1:2] + NT) \
            | (l80 == p[:, :, 2:3] + (NT + NY))
        oh = jnp.where(hit, 1.0, 0.0)[0].astype(jnp.bfloat16)
        cs = jax.lax.dot_general(
            oh, tab, (((1,), (0,)), ((), ())),
            preferred_element_type=jnp.float32)
        c = cs[None, :, :ROW]
        s = cs[None, :, ROW:]
        x = tokens_ref[:, sl, :]
        rl = jnp.concatenate([x[:, :, 16:], x[:, :, :16]], axis=-1)
        rr = jnp.concatenate([x[:, :, -16:], x[:, :, :-16]], axis=-1)
        lane = jax.lax.broadcasted_iota(jnp.int32, (1, 1, ROW), 2)
        r = jnp.where(lane % 32 < 16, rl, rr)
        out_ref[:, sl, :] = x * c + r * s


def _build_table(H):
    inv_freq = 1.0 / BASE ** (jnp.arange(0, 32, 2, dtype=jnp.float32) / 32.0)

    def sec_rows(n, lo, hi):
        th = jnp.arange(n, dtype=jnp.float32)[:, None] * inv_freq[None, :]
        out = []
        for f in (jnp.cos, jnp.sin):
            v = f(th)
            v32 = jnp.concatenate([v, v], axis=-1)
            row96 = jnp.concatenate(
                [jnp.zeros((n, lo), jnp.float32), v32,
                 jnp.zeros((n, hi), jnp.float32)], axis=-1)
            out.append(jnp.tile(row96, (1, H)))
        return jnp.concatenate(out, axis=-1)

    return jnp.concatenate([
        sec_rows(NT, 0, 64), sec_rows(NY, 32, 32), sec_rows(NX, 64, 0),
    ], axis=0)


@jax.jit
def kernel(tokens, pos_t, pos_y, pos_x):
    B, N, H, dim = tokens.shape
    pos = jnp.stack([pos_t, pos_y, pos_x], axis=-1).astype(jnp.int8)
    tok2 = tokens.reshape(B, N, H * dim)
    table = _build_table(H).astype(jnp.bfloat16)
    grid = (B, N // NB)
    out = pl.pallas_call(
        _rope_kernel,
        grid=grid,
        in_specs=[
            pl.BlockSpec((1, NB, 3), lambda b, i: (b, i, 0)),
            pl.BlockSpec((NT + NY + NX, 2 * H * dim), lambda b, i: (0, 0)),
            pl.BlockSpec((1, NB, H * dim), lambda b, i: (b, i, 0)),
        ],
        out_specs=pl.BlockSpec((1, NB, H * dim), lambda b, i: (b, i, 0)),
        out_shape=jax.ShapeDtypeStruct((B, N, H * dim), tokens.dtype),
        compiler_params=pltpu.CompilerParams(vmem_limit_bytes=66_000_000),
    )(pos, table, tok2)
    return out.reshape(B, N, H, dim)


# speedup vs baseline: 1.1219x; 1.1219x over previous
import jax
import jax.numpy as jnp
from jax.experimental import pallas as pl

BASE = 10000.0
NB = 1024
ROW = 1536
NT, NY, NX = 16, 32, 32


def _rope_kernel(pos_ref, tab_ref, tokens_ref, out_ref):
    p = pos_ref[...]
    l80 = jax.lax.broadcasted_iota(jnp.int32, (1, NB, NT + NY + NX), 2)
    hit = (l80 == p[:, :, 0:1]) | (l80 == p[:, :, 1:2] + NT) \
        | (l80 == p[:, :, 2:3] + (NT + NY))
    oh = jnp.where(hit, 1.0, 0.0)[0].astype(jnp.bfloat16)
    cs = jax.lax.dot_general(
        oh, tab_ref[...], (((1,), (0,)), ((), ())),
        preferred_element_type=jnp.float32)
    c = cs[None, :, :ROW]
    s = cs[None, :, ROW:]
    x = tokens_ref[...]
    rl = jnp.concatenate([x[:, :, 16:], x[:, :, :16]], axis=-1)
    rr = jnp.concatenate([x[:, :, -16:], x[:, :, :-16]], axis=-1)
    lane = jax.lax.broadcasted_iota(jnp.int32, (1, 1, ROW), 2)
    r = jnp.where(lane % 32 < 16, rl, rr)
    out_ref[...] = x * c + r * s


def _build_table(H):
    inv_freq = 1.0 / BASE ** (jnp.arange(0, 32, 2, dtype=jnp.float32) / 32.0)

    def sec_rows(n, lo, hi):
        th = jnp.arange(n, dtype=jnp.float32)[:, None] * inv_freq[None, :]
        out = []
        for f in (jnp.cos, jnp.sin):
            v = f(th)
            v32 = jnp.concatenate([v, v], axis=-1)
            row96 = jnp.concatenate(
                [jnp.zeros((n, lo), jnp.float32), v32,
                 jnp.zeros((n, hi), jnp.float32)], axis=-1)
            out.append(jnp.tile(row96, (1, H)))
        return jnp.concatenate(out, axis=-1)

    return jnp.concatenate([
        sec_rows(NT, 0, 64), sec_rows(NY, 32, 32), sec_rows(NX, 64, 0),
    ], axis=0)


@jax.jit
def kernel(tokens, pos_t, pos_y, pos_x):
    B, N, H, dim = tokens.shape
    pos = jnp.stack([pos_t, pos_y, pos_x], axis=-1)
    tok2 = tokens.reshape(B, N, H * dim)
    table = _build_table(H).astype(jnp.bfloat16)
    grid = (B, N // NB)
    out = pl.pallas_call(
        _rope_kernel,
        grid=grid,
        in_specs=[
            pl.BlockSpec((1, NB, 3), lambda b, i: (b, i, 0)),
            pl.BlockSpec((NT + NY + NX, 2 * H * dim), lambda b, i: (0, 0)),
            pl.BlockSpec((1, NB, H * dim), lambda b, i: (b, i, 0)),
        ],
        out_specs=pl.BlockSpec((1, NB, H * dim), lambda b, i: (b, i, 0)),
        out_shape=jax.ShapeDtypeStruct((B, N, H * dim), tokens.dtype),
    )(pos, table, tok2)
    return out.reshape(B, N, H, dim)
